# Initial kernel scaffold; baseline (speedup 1.0000x reference)
import jax
import jax.numpy as jnp
from jax.experimental import pallas as pl


def kernel(y, emb0, emb1):
    raise NotImplementedError("write your pallas kernel here")


# speedup vs baseline: 2.4698x; 2.4698x over previous
import dataclasses

import jax
import jax.numpy as jnp
from jax import lax
from jax.experimental import pallas as pl
from jax.experimental.pallas import tpu as pltpu
from jax.experimental.pallas import tpu_sc as plsc

_CONTEXT = 2
_BLANK = 100000
_D0, _D1 = 96, 32
_D = _D0 + _D1
_NC, _NS = 2, 16
_NW = _NC * _NS
_CHUNK = 128
_EPS = 1e-5
_L = 16


def _rsqrt16(x):
    h = x * 0.5
    i = plsc.bitcast(x, jnp.int32)
    g = plsc.bitcast(jnp.full((_L,), 0x5F3759DF, jnp.int32) - (i >> 1),
                     jnp.float32)
    for _ in range(3):
        g = g * (1.5 - h * g * g)
    return g


def _sc_body(y_hbm, ys_hbm, emb0_hbm, emb1_hbm, out_hbm,
             i0_v, i1_v, r0_v, r1_v, out_v, sem0, sem1):
    wid = lax.axis_index("s") * _NC + lax.axis_index("c")
    per_w = out_hbm.shape[0] // _NW
    n_chunks = per_w // _CHUNK

    @pl.loop(0, n_chunks)
    def _chunk(ci):
        base = wid * per_w + ci * _CHUNK
        pltpu.sync_copy(y_hbm.at[pl.ds(base, _CHUNK)], i0_v)
        pltpu.sync_copy(ys_hbm.at[pl.ds(base, _CHUNK)], i1_v)
        c0 = pltpu.async_copy(emb0_hbm.at[i0_v], r0_v, sem0)
        c1 = pltpu.async_copy(emb1_hbm.at[i1_v], r1_v, sem1)
        c0.wait()
        c1.wait()

        @pl.loop(0, _CHUNK)
        def _tok(t):
            vs = [r0_v[t, pl.ds(_L * j, _L)] for j in range(_D0 // _L)]
            vs += [r1_v[t, pl.ds(_L * j, _L)] for j in range(_D1 // _L)]
            s = ((vs[0] + vs[1]) + (vs[2] + vs[3])) + \
                ((vs[4] + vs[5]) + (vs[6] + vs[7]))
            q = ((vs[0] * vs[0] + vs[1] * vs[1]) +
                 (vs[2] * vs[2] + vs[3] * vs[3])) + \
                ((vs[4] * vs[4] + vs[5] * vs[5]) +
                 (vs[6] * vs[6] + vs[7] * vs[7]))
            mean = jnp.sum(s) * (1.0 / _D)
            var = jnp.sum(q) * (1.0 / _D) - mean * mean + _EPS
            r = _rsqrt16(jnp.full((_L,), var, jnp.float32))
            m = jnp.full((_L,), mean, jnp.float32)
            for j in range(_D // _L):
                out_v[t, pl.ds(_L * j, _L)] = (vs[j] - m) * r

        pltpu.sync_copy(out_v, out_hbm.at[pl.ds(base, _CHUNK)])


def kernel(y, emb0, emb1):
    B, U = y.shape
    n_tok = B * U
    y_flat = y.reshape(n_tok)
    y_shift = jnp.concatenate(
        [jnp.full((B, 1), _BLANK, jnp.int32), y[:, :-1]], axis=1
    ).reshape(n_tok)

    cp = pltpu.CompilerParams(
        needs_layout_passes=False, use_tc_tiling_on_sc=False)
    run = pl.kernel(
        _sc_body,
        compiler_params=cp,
        out_type=jax.ShapeDtypeStruct((n_tok, _D), jnp.float32),
        mesh=plsc.VectorSubcoreMesh(core_axis_name="c", subcore_axis_name="s"),
        scratch_types=[
            pltpu.VMEM((_CHUNK,), jnp.int32),
            pltpu.VMEM((_CHUNK,), jnp.int32),
            pltpu.VMEM((_CHUNK, _D0), jnp.float32),
            pltpu.VMEM((_CHUNK, _D1), jnp.float32),
            pltpu.VMEM((_CHUNK, _D), jnp.float32),
            pltpu.SemaphoreType.DMA,
            pltpu.SemaphoreType.DMA,
        ],
    )
    out = run(y_flat, y_shift, emb0, emb1).reshape(B, U, _D)
    state = y[:, U - _CONTEXT + 1:]
    return (out, state)


# speedup vs baseline: 3.6582x; 1.4812x over previous
import dataclasses

import jax
import jax.numpy as jnp
from jax import lax
from jax.experimental import pallas as pl
from jax.experimental.pallas import tpu as pltpu
from jax.experimental.pallas import tpu_sc as plsc

_CONTEXT = 2
_BLANK = 100000
_D0, _D1 = 96, 32
_D = _D0 + _D1
_NC, _NS = 2, 16
_NW = _NC * _NS
_CHUNK = 128
_EPS = 1e-5
_L = 16


def _rsqrt16(x):
    h = x * 0.5
    i = plsc.bitcast(x, jnp.int32)
    g = plsc.bitcast(jnp.full((_L,), 0x5F3759DF, jnp.int32) - (i >> 1),
                     jnp.float32)
    for _ in range(3):
        g = g * (1.5 - h * g * g)
    return g


def _sc_body(y_hbm, ys_hbm, emb0_hbm, emb1_hbm, out_hbm,
             i0_v, i1_v, r0_v, r1_v, out_v, sem0, sem1):
    wid = lax.axis_index("s") * _NC + lax.axis_index("c")
    per_w = out_hbm.shape[0] // _NW
    n_chunks = per_w // _CHUNK

    @pl.loop(0, n_chunks)
    def _chunk(ci):
        base = wid * per_w + ci * _CHUNK
        pltpu.sync_copy(y_hbm.at[pl.ds(base, _CHUNK)], i0_v)
        pltpu.sync_copy(ys_hbm.at[pl.ds(base, _CHUNK)], i1_v)
        c0 = pltpu.async_copy(emb0_hbm.at[i0_v], r0_v, sem0)
        c1 = pltpu.async_copy(emb1_hbm.at[i1_v], r1_v, sem1)
        c0.wait()
        c1.wait()

        @plsc.parallel_loop(0, _CHUNK, unroll=4)
        def _tok(t):
            vs = [r0_v[t, pl.ds(_L * j, _L)] for j in range(_D0 // _L)]
            vs += [r1_v[t, pl.ds(_L * j, _L)] for j in range(_D1 // _L)]
            s = ((vs[0] + vs[1]) + (vs[2] + vs[3])) + \
                ((vs[4] + vs[5]) + (vs[6] + vs[7]))
            q = ((vs[0] * vs[0] + vs[1] * vs[1]) +
                 (vs[2] * vs[2] + vs[3] * vs[3])) + \
                ((vs[4] * vs[4] + vs[5] * vs[5]) +
                 (vs[6] * vs[6] + vs[7] * vs[7]))
            mean = jnp.sum(s) * (1.0 / _D)
            var = jnp.sum(q) * (1.0 / _D) - mean * mean + _EPS
            r = _rsqrt16(jnp.full((_L,), var, jnp.float32))
            m = jnp.full((_L,), mean, jnp.float32)
            for j in range(_D // _L):
                out_v[t, pl.ds(_L * j, _L)] = (vs[j] - m) * r

        pltpu.sync_copy(out_v, out_hbm.at[pl.ds(base, _CHUNK)])


def kernel(y, emb0, emb1):
    B, U = y.shape
    n_tok = B * U
    y_flat = y.reshape(n_tok)
    y_shift = jnp.concatenate(
        [jnp.full((B, 1), _BLANK, jnp.int32), y[:, :-1]], axis=1
    ).reshape(n_tok)

    cp = pltpu.CompilerParams(
        needs_layout_passes=False, use_tc_tiling_on_sc=False)
    run = pl.kernel(
        _sc_body,
        compiler_params=cp,
        out_type=jax.ShapeDtypeStruct((n_tok, _D), jnp.float32),
        mesh=plsc.VectorSubcoreMesh(core_axis_name="c", subcore_axis_name="s"),
        scratch_types=[
            pltpu.VMEM((_CHUNK,), jnp.int32),
            pltpu.VMEM((_CHUNK,), jnp.int32),
            pltpu.VMEM((_CHUNK, _D0), jnp.float32),
            pltpu.VMEM((_CHUNK, _D1), jnp.float32),
            pltpu.VMEM((_CHUNK, _D), jnp.float32),
            pltpu.SemaphoreType.DMA,
            pltpu.SemaphoreType.DMA,
        ],
    )
    out = run(y_flat, y_shift, emb0, emb1).reshape(B, U, _D)
    state = y[:, U - _CONTEXT + 1:]
    return (out, state)


# speedup vs baseline: 5.0783x; 1.3882x over previous
import jax
import jax.numpy as jnp
from jax import lax
from jax.experimental import pallas as pl
from jax.experimental.pallas import tpu as pltpu
from jax.experimental.pallas import tpu_sc as plsc

_CONTEXT = 2
_BLANK = 100000
_D0, _D1 = 96, 32
_D = _D0 + _D1
_NC, _NS = 2, 16
_NW = _NC * _NS
_CHUNK = 128
_EPS = 1e-5
_L = 16


def _rsqrt16(x):
    h = x * 0.5
    i = plsc.bitcast(x, jnp.int32)
    g = plsc.bitcast(jnp.full((_L,), 0x5F3759DF, jnp.int32) - (i >> 1),
                     jnp.float32)
    for _ in range(3):
        g = g * (1.5 - h * g * g)
    return g


def _sc_body(y_hbm, ys_hbm, emb0_hbm, emb1_hbm, out_hbm,
             i0_all, i1_all, r0s, r1s, outs,
             g0a, g1a, g0b, g1b, w0, w1):
    wid = lax.axis_index("s") * _NC + lax.axis_index("c")
    per_w = out_hbm.shape[0] // _NW
    n_chunks = per_w // _CHUNK
    base_w = wid * per_w
    gsems = ((g0a, g1a), (g0b, g1b))
    wsems = (w0, w1)

    pltpu.sync_copy(y_hbm.at[pl.ds(base_w, per_w)], i0_all)
    pltpu.sync_copy(ys_hbm.at[pl.ds(base_w, per_w)], i1_all)

    def gather_pair(slot, ci):
        s0, s1 = gsems[slot]
        sl = pl.ds(ci * _CHUNK, _CHUNK)
        a = pltpu.make_async_copy(emb0_hbm.at[i0_all.at[sl]], r0s.at[slot], s0)
        b = pltpu.make_async_copy(emb1_hbm.at[i1_all.at[sl]], r1s.at[slot], s1)
        return a, b

    def fire(slot, ci):
        a, b = gather_pair(slot, ci)
        a.start()
        b.start()

    def wait_gathers(slot, ci):
        a, b = gather_pair(slot, ci)
        a.wait()
        b.wait()

    def out_copy(slot, ci):
        base = base_w + ci * _CHUNK
        return pltpu.make_async_copy(
            outs.at[slot], out_hbm.at[pl.ds(base, _CHUNK)], wsems[slot])

    def compute(slot, ci):
        r0_v = r0s.at[slot]
        r1_v = r1s.at[slot]
        out_v = outs.at[slot]

        @plsc.parallel_loop(0, _CHUNK, unroll=4)
        def _tok(t):
            vs = [r0_v[t, pl.ds(_L * j, _L)] for j in range(_D0 // _L)]
            vs += [r1_v[t, pl.ds(_L * j, _L)] for j in range(_D1 // _L)]
            s = ((vs[0] + vs[1]) + (vs[2] + vs[3])) + \
                ((vs[4] + vs[5]) + (vs[6] + vs[7]))
            q = ((vs[0] * vs[0] + vs[1] * vs[1]) +
                 (vs[2] * vs[2] + vs[3] * vs[3])) + \
                ((vs[4] * vs[4] + vs[5] * vs[5]) +
                 (vs[6] * vs[6] + vs[7] * vs[7]))
            mean = jnp.sum(s) * (1.0 / _D)
            var = jnp.sum(q) * (1.0 / _D) - mean * mean + _EPS
            r = _rsqrt16(jnp.full((_L,), var, jnp.float32))
            m = jnp.full((_L,), mean, jnp.float32)
            for j in range(_D // _L):
                out_v[t, pl.ds(_L * j, _L)] = (vs[j] - m) * r

    fire(0, 0)

    @pl.loop(0, n_chunks, step=2)
    def _pair(c):
        fire(1, c + 1)
        wait_gathers(0, c)

        @pl.when(c >= 2)
        def _():
            out_copy(0, c).wait()
        compute(0, c)
        out_copy(0, c).start()

        @pl.when(c + 2 < n_chunks)
        def _():
            fire(0, c + 2)
        wait_gathers(1, c + 1)

        @pl.when(c >= 2)
        def _():
            out_copy(1, c + 1).wait()
        compute(1, c + 1)
        out_copy(1, c + 1).start()

    out_copy(0, n_chunks - 2).wait()
    out_copy(1, n_chunks - 1).wait()


def kernel(y, emb0, emb1):
    B, U = y.shape
    n_tok = B * U
    per_w = n_tok // _NW
    y_flat = y.reshape(n_tok)
    y_shift = jnp.concatenate(
        [jnp.full((B, 1), _BLANK, jnp.int32), y[:, :-1]], axis=1
    ).reshape(n_tok)

    cp = pltpu.CompilerParams(
        needs_layout_passes=False, use_tc_tiling_on_sc=False)
    run = pl.kernel(
        _sc_body,
        compiler_params=cp,
        out_type=jax.ShapeDtypeStruct((n_tok, _D), jnp.float32),
        mesh=plsc.VectorSubcoreMesh(core_axis_name="c", subcore_axis_name="s"),
        scratch_types=[
            pltpu.VMEM((per_w,), jnp.int32),
            pltpu.VMEM((per_w,), jnp.int32),
            pltpu.VMEM((2, _CHUNK, _D0), jnp.float32),
            pltpu.VMEM((2, _CHUNK, _D1), jnp.float32),
            pltpu.VMEM((2, _CHUNK, _D), jnp.float32),
            pltpu.SemaphoreType.DMA,
            pltpu.SemaphoreType.DMA,
            pltpu.SemaphoreType.DMA,
            pltpu.SemaphoreType.DMA,
            pltpu.SemaphoreType.DMA,
            pltpu.SemaphoreType.DMA,
        ],
    )
    out = run(y_flat, y_shift, emb0, emb1).reshape(B, U, _D)
    state = y[:, U - _CONTEXT + 1:]
    return (out, state)


# speedup vs baseline: 5.1052x; 1.0053x over previous
import jax
import jax.numpy as jnp
from jax import lax
from jax.experimental import pallas as pl
from jax.experimental.pallas import tpu as pltpu
from jax.experimental.pallas import tpu_sc as plsc

_CONTEXT = 2
_D0, _D1 = 96, 32
_D = _D0 + _D1
_NC, _NS = 2, 16
_NW = _NC * _NS
_CHUNK = 128
_EPS = 1e-5
_L = 16


def _rsqrt16(x):
    h = x * 0.5
    i = plsc.bitcast(x, jnp.int32)
    g = plsc.bitcast(jnp.full((_L,), 0x5F3759DF, jnp.int32) - (i >> 1),
                     jnp.float32)
    for _ in range(3):
        g = g * (1.5 - h * g * g)
    return g


def _make_body(U):
    def _sc_body(y_hbm, comb_hbm, out_hbm,
                 i_all, gs, outs, tails, g0, g1, w0, w1):
        wid = lax.axis_index("s") * _NC + lax.axis_index("c")
        per_w = out_hbm.shape[0] // _NW
        n_chunks = per_w // _CHUNK
        base_w = wid * per_w
        gsems = (g0, g1)
        wsems = (w0, w1)

        pltpu.sync_copy(y_hbm.at[pl.ds(base_w, per_w)], i_all)

        def gather(slot, ci):
            sl = pl.ds(ci * _CHUNK, _CHUNK)
            return pltpu.make_async_copy(
                comb_hbm.at[i_all.at[sl]], gs.at[slot], gsems[slot])

        def save_tail(slot):
            tails[slot, pl.ds(0, _L)] = gs[slot, _CHUNK - 1, pl.ds(_D0, _L)]
            tails[slot, pl.ds(_L, _L)] = \
                gs[slot, _CHUNK - 1, pl.ds(_D0 + _L, _L)]

        def out_copy(slot, ci):
            base = base_w + ci * _CHUNK
            return pltpu.make_async_copy(
                outs.at[slot], out_hbm.at[pl.ds(base, _CHUNK)], wsems[slot])

        def compute(slot, ci):
            g = gs.at[slot]
            out_v = outs.at[slot]
            base = base_w + ci * _CHUNK

            @plsc.parallel_loop(0, _CHUNK, unroll=4)
            def _tok(t):
                vs = [g[t, pl.ds(_L * j, _L)] for j in range(_D0 // _L)]
                tp = jnp.maximum(t - 1, 0)
                tv = jnp.full((_L,), t, jnp.int32)
                first = tv == 0
                e1a = jnp.where(first, tails[1 - slot, pl.ds(0, _L)],
                                g[tp, pl.ds(_D0, _L)])
                e1b = jnp.where(first, tails[1 - slot, pl.ds(_L, _L)],
                                g[tp, pl.ds(_D0 + _L, _L)])
                rem = lax.rem(base + t, U)
                row0 = jnp.full((_L,), rem, jnp.int32) == 0
                vs.append(jnp.where(row0, 0.0, e1a))
                vs.append(jnp.where(row0, 0.0, e1b))

                s = ((vs[0] + vs[1]) + (vs[2] + vs[3])) + \
                    ((vs[4] + vs[5]) + (vs[6] + vs[7]))
                q = ((vs[0] * vs[0] + vs[1] * vs[1]) +
                     (vs[2] * vs[2] + vs[3] * vs[3])) + \
                    ((vs[4] * vs[4] + vs[5] * vs[5]) +
                     (vs[6] * vs[6] + vs[7] * vs[7]))
                mean = jnp.sum(s) * (1.0 / _D)
                var = jnp.sum(q) * (1.0 / _D) - mean * mean + _EPS
                r = _rsqrt16(jnp.full((_L,), var, jnp.float32))
                m = jnp.full((_L,), mean, jnp.float32)
                for j in range(_D // _L):
                    out_v[t, pl.ds(_L * j, _L)] = (vs[j] - m) * r

        gather(0, 0).start()

        @pl.loop(0, n_chunks, step=2)
        def _pair(c):
            gather(1, c + 1).start()
            gather(0, c).wait()
            save_tail(0)

            @pl.when(c >= 2)
            def _():
                out_copy(0, c).wait()
            compute(0, c)
            out_copy(0, c).start()

            @pl.when(c + 2 < n_chunks)
            def _():
                gather(0, c + 2).start()
            gather(1, c + 1).wait()
            save_tail(1)

            @pl.when(c >= 2)
            def _():
                out_copy(1, c + 1).wait()
            compute(1, c + 1)
            out_copy(1, c + 1).start()

        out_copy(0, n_chunks - 2).wait()
        out_copy(1, n_chunks - 1).wait()

    return _sc_body


def kernel(y, emb0, emb1):
    B, U = y.shape
    n_tok = B * U
    per_w = n_tok // _NW
    y_flat = y.reshape(n_tok)
    comb = jnp.concatenate([emb0, emb1], axis=1)

    cp = pltpu.CompilerParams(
        needs_layout_passes=False, use_tc_tiling_on_sc=False)
    run = pl.kernel(
        _make_body(U),
        compiler_params=cp,
        out_type=jax.ShapeDtypeStruct((n_tok, _D), jnp.float32),
        mesh=plsc.VectorSubcoreMesh(core_axis_name="c", subcore_axis_name="s"),
        scratch_types=[
            pltpu.VMEM((per_w,), jnp.int32),
            pltpu.VMEM((2, _CHUNK, _D), jnp.float32),
            pltpu.VMEM((2, _CHUNK, _D), jnp.float32),
            pltpu.VMEM((2, 2 * _L), jnp.float32),
            pltpu.SemaphoreType.DMA,
            pltpu.SemaphoreType.DMA,
            pltpu.SemaphoreType.DMA,
            pltpu.SemaphoreType.DMA,
        ],
    )
    out = run(y_flat, comb).reshape(B, U, _D)
    state = y[:, U - _CONTEXT + 1:]
    return (out, state)


# speedup vs baseline: 5.1137x; 1.0017x over previous
import jax
import jax.numpy as jnp
from jax import lax
from jax.experimental import pallas as pl
from jax.experimental.pallas import tpu as pltpu
from jax.experimental.pallas import tpu_sc as plsc

_CONTEXT = 2
_D0, _D1 = 96, 32
_D = _D0 + _D1
_NC, _NS = 2, 16
_NW = _NC * _NS
_CHUNK = 128
_EPS = 1e-5
_L = 16


def _rsqrt16(x):
    h = x * 0.5
    i = plsc.bitcast(x, jnp.int32)
    g = plsc.bitcast(jnp.full((_L,), 0x5F3759DF, jnp.int32) - (i >> 1),
                     jnp.float32)
    for _ in range(3):
        g = g * (1.5 - h * g * g)
    return g


def _make_body(U):
    def _sc_body(y_hbm, comb_hbm, out_hbm,
                 i_all, gs, outs, tails, g0, g1, w0, w1):
        wid = lax.axis_index("s") * _NC + lax.axis_index("c")
        per_w = out_hbm.shape[0] // _NW
        n_chunks = per_w // _CHUNK
        base_w = wid * per_w
        gsems = (g0, g1)
        wsems = (w0, w1)

        pltpu.sync_copy(y_hbm.at[pl.ds(base_w, per_w)], i_all)

        def gather(slot, ci):
            sl = pl.ds(ci * _CHUNK, _CHUNK)
            return pltpu.make_async_copy(
                comb_hbm.at[i_all.at[sl]], gs.at[slot], gsems[slot])

        def save_tail(slot):
            tails[slot, pl.ds(0, _L)] = gs[slot, _CHUNK - 1, pl.ds(_D0, _L)]
            tails[slot, pl.ds(_L, _L)] = \
                gs[slot, _CHUNK - 1, pl.ds(_D0 + _L, _L)]

        def out_copy(slot, ci):
            base = base_w + ci * _CHUNK
            return pltpu.make_async_copy(
                outs.at[slot], out_hbm.at[pl.ds(base, _CHUNK)], wsems[slot])

        def compute(slot, ci):
            g = gs.at[slot]
            out_v = outs.at[slot]
            base = base_w + ci * _CHUNK

            @plsc.parallel_loop(0, _CHUNK, unroll=4)
            def _tok(t):
                vs = [g[t, pl.ds(_L * j, _L)] for j in range(_D0 // _L)]
                tp = jnp.maximum(t - 1, 0)
                tv = jnp.full((_L,), t, jnp.int32)
                first = tv == 0
                e1a = jnp.where(first, tails[1 - slot, pl.ds(0, _L)],
                                g[tp, pl.ds(_D0, _L)])
                e1b = jnp.where(first, tails[1 - slot, pl.ds(_L, _L)],
                                g[tp, pl.ds(_D0 + _L, _L)])
                rem = lax.rem(base + t, U)
                row0 = jnp.full((_L,), rem, jnp.int32) == 0
                vs.append(jnp.where(row0, 0.0, e1a))
                vs.append(jnp.where(row0, 0.0, e1b))

                s = ((vs[0] + vs[1]) + (vs[2] + vs[3])) + \
                    ((vs[4] + vs[5]) + (vs[6] + vs[7]))
                q = ((vs[0] * vs[0] + vs[1] * vs[1]) +
                     (vs[2] * vs[2] + vs[3] * vs[3])) + \
                    ((vs[4] * vs[4] + vs[5] * vs[5]) +
                     (vs[6] * vs[6] + vs[7] * vs[7]))
                mean = jnp.sum(s) * (1.0 / _D)
                var = jnp.sum(q) * (1.0 / _D) - mean * mean + _EPS
                r = _rsqrt16(jnp.full((_L,), var, jnp.float32))
                m = jnp.full((_L,), mean, jnp.float32)
                for j in range(_D // _L):
                    out_v[t, pl.ds(_L * j, _L)] = (vs[j] - m) * r

        gather(0, 0).start()

        @pl.loop(0, n_chunks, step=2)
        def _pair(c):
            gather(1, c + 1).start()
            gather(0, c).wait()
            save_tail(0)

            @pl.when(c >= 2)
            def _():
                out_copy(0, c).wait()
            compute(0, c)
            out_copy(0, c).start()

            @pl.when(c + 2 < n_chunks)
            def _():
                gather(0, c + 2).start()
            gather(1, c + 1).wait()
            save_tail(1)

            @pl.when(c >= 2)
            def _():
                out_copy(1, c + 1).wait()
            compute(1, c + 1)
            out_copy(1, c + 1).start()

        out_copy(0, n_chunks - 2).wait()
        out_copy(1, n_chunks - 1).wait()

    return _sc_body


def kernel(y, emb0, emb1):
    B, U = y.shape
    n_tok = B * U
    per_w = n_tok // _NW
    y_flat = y.reshape(n_tok)
    comb = jnp.concatenate([emb0, emb1], axis=1)

    cp = pltpu.CompilerParams(
        needs_layout_passes=False, use_tc_tiling_on_sc=True)
    run = pl.kernel(
        _make_body(U),
        compiler_params=cp,
        out_type=jax.ShapeDtypeStruct((n_tok, _D), jnp.float32),
        mesh=plsc.VectorSubcoreMesh(core_axis_name="c", subcore_axis_name="s"),
        scratch_types=[
            pltpu.VMEM((per_w,), jnp.int32),
            pltpu.VMEM((2, _CHUNK, _D), jnp.float32),
            pltpu.VMEM((2, _CHUNK, _D), jnp.float32),
            pltpu.VMEM((2, 2 * _L), jnp.float32),
            pltpu.SemaphoreType.DMA,
            pltpu.SemaphoreType.DMA,
            pltpu.SemaphoreType.DMA,
            pltpu.SemaphoreType.DMA,
        ],
    )
    out = run(y_flat, comb).reshape(B, U, _D)
    state = y[:, U - _CONTEXT + 1:]
    return (out, state)
